# baseline (speedup 1.0000x reference)
import functools
import math

import jax
import jax.numpy as jnp
import numpy as np
from jax import lax
from jax.experimental import pallas as pl
from jax.experimental.pallas import tpu as pltpu
from jax.experimental.pallas import tpu_sc as plsc

NUM_EMB = 100000
DIM = 32
BATCH = 16384
HIST = 200

NC = 2
NS = 16
NW = NC * NS

B_TOT = BATCH * HIST
B_PER_W = B_TOT // NW
TB = 2048
CHUNK = 1024
KSUB = CHUNK // 128
N_CHUNK = B_PER_W // CHUNK
NBUF = 2
assert B_PER_W * NW == B_TOT and N_CHUNK * CHUNK == B_PER_W
assert B_PER_W % TB == 0 and N_CHUNK % NBUF == 0


def _pe_host() -> np.ndarray:
    position = np.arange(0, NUM_EMB, dtype=np.float32)[:, None]
    div_term = np.exp(
        np.arange(0, DIM, 2, dtype=np.float32) * (-(math.log(10000.0) / DIM))
    )
    pe = np.zeros((NUM_EMB, DIM), dtype=np.float32)
    pe[:, 0::2] = np.sin(position * div_term)
    pe[:, 1::2] = np.cos(position * div_term)
    return pe


_PE = _pe_host()


def _add_body(wt_ref, pe_ref, out_ref):
    out_ref[...] = wt_ref[...].T + pe_ref[...]


_TBM = 2048


def _table_add(emb_weight):
    wt = emb_weight.T
    pe = jnp.asarray(_PE)
    grid = (NUM_EMB + _TBM - 1) // _TBM
    return pl.pallas_call(
        _add_body,
        out_shape=jax.ShapeDtypeStruct((NUM_EMB, DIM), jnp.float32),
        grid=(grid,),
        in_specs=[
            pl.BlockSpec((DIM, _TBM), lambda i: (0, i)),
            pl.BlockSpec((_TBM, DIM), lambda i: (i, 0)),
        ],
        out_specs=pl.BlockSpec((_TBM, DIM), lambda i: (i, 0)),
    )(wt, pe)


def _gather_body(table_hbm, idx_hbm, out_hbm, idx_v, rows_v,
                 idx_s0, idx_s1, gat_s0, gat_s1, out_s0, out_s1):
    wid = lax.axis_index("s") * NC + lax.axis_index("c")
    base = wid * (B_PER_W // 128)
    idx_sems = (idx_s0, idx_s1)
    gat_sems = (gat_s0, gat_s1)
    out_sems = (out_s0, out_s1)

    def idx_load(b, g):
        return pltpu.make_async_copy(
            idx_hbm.at[pl.ds(base + g * KSUB, KSUB)], idx_v.at[b],
            idx_sems[b])

    def gathers(b):
        return [
            pltpu.make_async_copy(
                table_hbm.at[idx_v.at[b, j]], rows_v.at[b, j], gat_sems[b])
            for j in range(KSUB)
        ]

    def writeback(b, g):
        return pltpu.make_async_copy(
            rows_v.at[b], out_hbm.at[pl.ds(base + g * KSUB, KSUB)],
            out_sems[b])

    for b in range(NBUF):
        idx_load(b, b).start()

    def outer(t, carry):
        go = t * NBUF
        for b in range(NBUF):
            g = go + b
            idx_load(b, g).wait()

            @pl.when(g >= NBUF)
            def _():
                writeback(b, 0).wait()

            for c in gathers(b):
                c.start()
        for b in range(NBUF):
            g = go + b
            for c in gathers(b):
                c.wait()

            @pl.when(g + NBUF < N_CHUNK)
            def _():
                idx_load(b, g + NBUF).start()

            writeback(b, g).start()
        return carry

    lax.fori_loop(0, N_CHUNK // NBUF, outer, 0)

    for b in range(NBUF):
        writeback(b, 0).wait()


def _gather_sc(table, idx_flat):
    k = functools.partial(
        pl.kernel,
        mesh=plsc.VectorSubcoreMesh(core_axis_name="c", subcore_axis_name="s"),
        out_type=jax.ShapeDtypeStruct((B_TOT // 128, 128, DIM), jnp.float32),
        scratch_types=[
            pltpu.VMEM((NBUF, KSUB, 128), jnp.int32),
            pltpu.VMEM((NBUF, KSUB, 128, DIM), jnp.float32),
            pltpu.SemaphoreType.DMA,
            pltpu.SemaphoreType.DMA,
            pltpu.SemaphoreType.DMA,
            pltpu.SemaphoreType.DMA,
            pltpu.SemaphoreType.DMA,
            pltpu.SemaphoreType.DMA,
        ],
        compiler_params=pltpu.CompilerParams(use_tc_tiling_on_sc=False),
    )(_gather_body)
    return k(table, idx_flat)


def _tr_body(in_ref, out_ref):
    x = in_ref[...]
    xt = x.T
    for q in range(4):
        out_ref[0, :, pl.ds(q * (TB // 4), TB // 4)] = (
            xt[32 * q:32 * (q + 1), :])


def _transpose_tc(m128):
    return pl.pallas_call(
        _tr_body,
        out_shape=jax.ShapeDtypeStruct((HIST, DIM, BATCH), jnp.float32),
        grid=(HIST, BATCH // TB),
        in_specs=[pl.BlockSpec((TB // 4, 128),
                               lambda h, b: (h * (BATCH // TB) + b, 0))],
        out_specs=pl.BlockSpec((1, DIM, TB), lambda h, b: (h, 0, b)),
    )(m128)


def kernel(indices, emb_weight):
    table = _table_add(emb_weight)
    idx_flat = indices.T.reshape(B_TOT).astype(jnp.int32)
    idx_flat = (idx_flat.reshape(B_TOT // TB, 4, TB // 4)
                .swapaxes(1, 2).reshape(B_TOT))
    m = _gather_sc(table, idx_flat.reshape(B_TOT // 128, 128))
    t3 = _transpose_tc(m.reshape(B_TOT // 4, 128))
    return jnp.transpose(t3, (2, 0, 1))


# speedup vs baseline: 1.7243x; 1.7243x over previous
import functools
import math

import jax
import jax.numpy as jnp
import numpy as np
from jax import lax
from jax.experimental import pallas as pl
from jax.experimental.pallas import tpu as pltpu
from jax.experimental.pallas import tpu_sc as plsc

NUM_EMB = 100000
DIM = 32
BATCH = 16384
HIST = 200

NC = 2
NS = 16
NW = NC * NS

B_TOT = BATCH * HIST
B_PER_W = B_TOT // NW
TB = 2048
CHUNK = 1024
KSUB = CHUNK // 128
N_CHUNK = B_PER_W // CHUNK
NBUF = 2
assert B_PER_W * NW == B_TOT and N_CHUNK * CHUNK == B_PER_W
assert B_PER_W % TB == 0 and N_CHUNK % NBUF == 0


def _pe_host() -> np.ndarray:
    position = np.arange(0, NUM_EMB, dtype=np.float32)[:, None]
    div_term = np.exp(
        np.arange(0, DIM, 2, dtype=np.float32) * (-(math.log(10000.0) / DIM))
    )
    pe = np.zeros((NUM_EMB, DIM), dtype=np.float32)
    pe[:, 0::2] = np.sin(position * div_term)
    pe[:, 1::2] = np.cos(position * div_term)
    return pe


_PE = _pe_host()


def _add_body(wt_ref, pe_ref, out_ref):
    out_ref[...] = wt_ref[...].T + pe_ref[...]


_TBM = 2048


def _table_add(emb_weight):
    wt = emb_weight.T
    pe = jnp.asarray(_PE)
    grid = (NUM_EMB + _TBM - 1) // _TBM
    return pl.pallas_call(
        _add_body,
        out_shape=jax.ShapeDtypeStruct((NUM_EMB, DIM), jnp.float32),
        grid=(grid,),
        in_specs=[
            pl.BlockSpec((DIM, _TBM), lambda i: (0, i)),
            pl.BlockSpec((_TBM, DIM), lambda i: (i, 0)),
        ],
        out_specs=pl.BlockSpec((_TBM, DIM), lambda i: (i, 0)),
    )(wt, pe)


def _gather_body(table_hbm, idx_hbm, out_hbm, idx_v, rows_v,
                 idx_s0, idx_s1, gat_s0, gat_s1, out_s0, out_s1):
    wid = lax.axis_index("s") * NC + lax.axis_index("c")
    base_i = wid * (B_PER_W // 128)
    base_a = wid * (B_PER_W // 4)
    idx_sems = (idx_s0, idx_s1)
    gat_sems = (gat_s0, gat_s1)
    out_sems = (out_s0, out_s1)

    def idx_load(b, g):
        return pltpu.make_async_copy(
            idx_hbm.at[pl.ds(base_i + g * KSUB, KSUB)], idx_v.at[b],
            idx_sems[b])

    def gathers(b):
        return [
            pltpu.make_async_copy(
                table_hbm.at[idx_v.at[b, j]],
                rows_v.at[b, j // 4, pl.ds((j % 4) * 128, 128)],
                gat_sems[b])
            for j in range(KSUB)
        ]

    def writeback(b, t):
        c0 = 2 * (b % 2)
        a0 = base_a + t * (TB // 4)
        return [
            pltpu.make_async_copy(
                rows_v.at[b, s],
                out_hbm.at[pl.ds(a0, TB // 4),
                           pl.ds((c0 + s) * DIM, DIM)],
                out_sems[b])
            for s in range(2)
        ]

    for b in range(NBUF):
        idx_load(b, b).start()

    def outer(t, carry):
        go = t * NBUF
        for b in range(NBUF):
            g = go + b
            idx_load(b, g).wait()

            @pl.when(g >= NBUF)
            def _():
                for c in writeback(b, 0):
                    c.wait()

            for c in gathers(b):
                c.start()
        for b in range(NBUF):
            g = go + b
            for c in gathers(b):
                c.wait()

            @pl.when(g + NBUF < N_CHUNK)
            def _():
                idx_load(b, g + NBUF).start()

            for c in writeback(b, t):
                c.start()
        return carry

    lax.fori_loop(0, N_CHUNK // NBUF, outer, 0)

    for b in range(NBUF):
        for c in writeback(b, 0):
            c.wait()


def _gather_sc(table, idx_flat):
    k = functools.partial(
        pl.kernel,
        mesh=plsc.VectorSubcoreMesh(core_axis_name="c", subcore_axis_name="s"),
        out_type=jax.ShapeDtypeStruct((B_TOT // 4, 4 * DIM), jnp.float32),
        scratch_types=[
            pltpu.VMEM((NBUF, KSUB, 128), jnp.int32),
            pltpu.VMEM((NBUF, 2, CHUNK // 2, DIM), jnp.float32),
            pltpu.SemaphoreType.DMA,
            pltpu.SemaphoreType.DMA,
            pltpu.SemaphoreType.DMA,
            pltpu.SemaphoreType.DMA,
            pltpu.SemaphoreType.DMA,
            pltpu.SemaphoreType.DMA,
        ],
        compiler_params=pltpu.CompilerParams(use_tc_tiling_on_sc=False),
    )(_gather_body)
    return k(table, idx_flat)


def _tr_body(in_ref, out_ref):
    x = in_ref[...]
    xt = x.T
    for q in range(4):
        out_ref[0, :, pl.ds(q * (TB // 4), TB // 4)] = (
            xt[32 * q:32 * (q + 1), :])


def _transpose_tc(m128):
    return pl.pallas_call(
        _tr_body,
        out_shape=jax.ShapeDtypeStruct((HIST, DIM, BATCH), jnp.float32),
        grid=(HIST, BATCH // TB),
        in_specs=[pl.BlockSpec((TB // 4, 128),
                               lambda h, b: (h * (BATCH // TB) + b, 0))],
        out_specs=pl.BlockSpec((1, DIM, TB), lambda h, b: (h, 0, b)),
    )(m128)


def kernel(indices, emb_weight):
    table = _table_add(emb_weight)
    idx_flat = indices.T.reshape(B_TOT).astype(jnp.int32)
    m = _gather_sc(table, idx_flat.reshape(B_TOT // 128, 128))
    t3 = _transpose_tc(m)
    return jnp.transpose(t3, (2, 0, 1))
